# baseline (speedup 1.0000x reference)
import functools

import jax
import jax.numpy as jnp
from jax import lax
from jax.experimental import pallas as pl
from jax.experimental.pallas import tpu as pltpu
from jax.experimental.pallas import tpu_sc as plsc

_K = 4
_N = 10000
_E = 320000
_D = 128
_DS = _D // _K

_NC = 2
_NS = 16
_NW = _NC * _NS
_CH = 128
_W = 8
_NWIN = 10
_C = _NWIN * _W
_EPAD = _NW * _C * _CH
_NACC = 10112
_RPT = _NACC // _NS


def _sc_scatter_add(x, srcp, dstp, zeros_tile):
    mesh = plsc.VectorSubcoreMesh(core_axis_name="c", subcore_axis_name="s")

    @functools.partial(
        pl.kernel,
        out_type=jax.ShapeDtypeStruct((_NC * _NACC, _D), jnp.float32),
        mesh=mesh,
        scratch_types=(
            [pltpu.VMEM((_W, _CH), jnp.int32) for _ in range(4)]
            + [pltpu.VMEM((_CH, _D), jnp.float32) for _ in range(2)]
            + [pltpu.VMEM_SHARED((_NACC, _D), jnp.float32)]
            + [pltpu.SemaphoreType.DMA for _ in range(6)]),
    )
    def k(x_hbm, src_hbm, dst_hbm, z_hbm, out_hbm, *rest):
        swin = rest[0:2]
        dwin = rest[2:4]
        rows = rest[4:6]
        acc = rest[6]
        wsem = rest[7:9]
        gsem = rest[9:11]
        ssem = rest[11:13]
        c = lax.axis_index("c")
        s = lax.axis_index("s")
        wid = c * _NS + s
        wbase = wid * (_NWIN + 1)
        pltpu.sync_copy(z_hbm, acc.at[pl.ds(s * _RPT, _RPT)])
        plsc.subcore_barrier()

        def fetch(w, wb):
            pltpu.async_copy(src_hbm.at[wbase + w], swin[wb], wsem[wb])
            pltpu.async_copy(dst_hbm.at[wbase + w], dwin[wb], wsem[wb])

        def fetch_wait(w, wb):
            pltpu.make_async_copy(src_hbm.at[wbase + w], swin[wb],
                                  wsem[wb]).wait()
            pltpu.make_async_copy(dst_hbm.at[wbase + w], dwin[wb],
                                  wsem[wb]).wait()

        fetch(0, 0)
        fetch_wait(0, 0)

        def window(w, wb):
            fetch(w + 1, 1 - wb)
            for kk in range(_W):
                b = kk % 2
                if kk >= 2:
                    pltpu.make_async_copy(
                        rows[b], acc.at[dwin[wb].at[kk - 2]],
                        ssem[b]).wait()
                pltpu.async_copy(x_hbm.at[swin[wb].at[kk]], rows[b],
                                 gsem[b]).wait()
                pltpu.async_copy(rows[b], acc.at[dwin[wb].at[kk]],
                                 ssem[b], add=True)
            for kk in (_W - 2, _W - 1):
                pltpu.make_async_copy(rows[kk % 2],
                                      acc.at[dwin[wb].at[kk]],
                                      ssem[kk % 2]).wait()
            fetch_wait(w + 1, 1 - wb)

        def step(t, carry):
            window(2 * t, 0)
            window(2 * t + 1, 1)
            return carry

        lax.fori_loop(0, _NWIN // 2, step, 0)
        plsc.subcore_barrier()
        pltpu.sync_copy(acc.at[pl.ds(s * _RPT, _RPT)],
                        out_hbm.at[pl.ds(c * _NACC + s * _RPT, _RPT)])

    return k(x, srcp, dstp, zeros_tile)


def _mlp_body(scale_ref, x_ref, a_ref, b_ref, w1_ref, c1_ref, w2_ref,
              c2_ref, wl_ref, cl_ref, o_ref):
    h = x_ref[...] + a_ref[...] + b_ref[...]
    r = jnp.maximum(
        jnp.dot(h, w1_ref[...], preferred_element_type=jnp.float32)
        + c1_ref[...], 0.0)
    z = (jnp.dot(r, w2_ref[...], preferred_element_type=jnp.float32)
         + c2_ref[...])
    lin = (jnp.dot(x_ref[...], wl_ref[...],
                   preferred_element_type=jnp.float32) + cl_ref[...])
    o_ref[...] = z + scale_ref[0, 0] * lin


def _block_diag(w):
    k, a, b = w.shape
    out = jnp.zeros((k * a, k * b), w.dtype)
    for i in range(k):
        out = out.at[i * a:(i + 1) * a, i * b:(i + 1) * b].set(w[i])
    return out


def kernel(x, edge_index, edge_weights, edge_attr, with_linear, W1, b1, W2,
           b2, Wl, bl):
    src = edge_index[0]
    dst = edge_index[1]
    pad = _EPAD - _E
    per_tile = _E // _NW
    tpad = _C * _CH - per_tile
    psrc = jnp.zeros((_NW, tpad), jnp.int32)
    pdst = jnp.broadcast_to(
        _N + (jnp.arange(tpad, dtype=jnp.int32) % (_NACC - _N)),
        (_NW, tpad))
    srcp = jnp.concatenate([src.reshape(_NW, per_tile), psrc], axis=1)
    dstp = jnp.concatenate([dst.reshape(_NW, per_tile), pdst], axis=1)
    srcp = srcp.reshape(_NW, _NWIN, _W, _CH)
    dstp = dstp.reshape(_NW, _NWIN, _W, _CH)
    zwin = jnp.zeros((_NW, 1, _W, _CH), jnp.int32)
    srcp = jnp.concatenate([srcp, zwin], axis=1).reshape(
        _NW * (_NWIN + 1), _W, _CH)
    dstp = jnp.concatenate([dstp, zwin], axis=1).reshape(
        _NW * (_NWIN + 1), _W, _CH)
    zeros_tile = jnp.zeros((_RPT, _D), jnp.float32)

    parts = _sc_scatter_add(x, srcp, dstp, zeros_tile)
    agg0 = parts[:_N]
    agg1 = parts[_NACC:_NACC + _N]

    bd1 = _block_diag(W1)
    bd2 = _block_diag(W2)
    bdl = _block_diag(Wl[:, :_DS, :_DS])
    c1 = b1.reshape(1, _D)
    c2 = b2.reshape(1, _D)
    cl = bl[:, :_DS].reshape(1, _D)
    scale = jnp.where(with_linear != 0, 1.0, 0.0).astype(
        jnp.float32).reshape(1, 1)

    blk = 1000
    grid = _N // blk
    full = pl.BlockSpec((_D, _D), lambda i: (0, 0))
    bias = pl.BlockSpec((1, _D), lambda i: (0, 0))
    rows = pl.BlockSpec((blk, _D), lambda i: (i, 0))
    out = pl.pallas_call(
        _mlp_body,
        grid=(grid,),
        in_specs=[pl.BlockSpec((1, 1), lambda i: (0, 0)),
                  rows, rows, rows, full, bias, full, bias, full, bias],
        out_specs=rows,
        out_shape=jax.ShapeDtypeStruct((_N, _D), jnp.float32),
    )(scale, x, agg0, agg1, bd1, c1, bd2, c2, bdl, cl)
    return out


# speedup vs baseline: 1.3234x; 1.3234x over previous
import functools

import jax
import jax.numpy as jnp
from jax import lax
from jax.experimental import pallas as pl
from jax.experimental.pallas import tpu as pltpu
from jax.experimental.pallas import tpu_sc as plsc

_K = 4
_N = 10000
_E = 320000
_D = 128
_DS = _D // _K

_NC = 2
_NS = 16
_NW = _NC * _NS
_CH = 128
_C = 79
_EPAD = _NW * _C * _CH
_NACC = 10112
_RPT = _NACC // _NS


def _sc_scatter_add(x, srcp, dstp, zeros_tile):
    mesh = plsc.VectorSubcoreMesh(core_axis_name="c", subcore_axis_name="s")

    @functools.partial(
        pl.kernel,
        out_type=jax.ShapeDtypeStruct((_NC * _NACC, _D), jnp.float32),
        mesh=mesh,
        scratch_types=(
            [pltpu.VMEM((_C, _CH), jnp.int32),
             pltpu.VMEM((_C, _CH), jnp.int32),
             pltpu.VMEM((_CH, _D), jnp.float32),
             pltpu.VMEM_SHARED((_NACC, _D), jnp.float32),
             pltpu.SemaphoreType.DMA]),
    )
    def k(x_hbm, src_hbm, dst_hbm, z_hbm, out_hbm, src_v, dst_v, rows_v,
          acc, sem):
        c = lax.axis_index("c")
        s = lax.axis_index("s")
        wid = c * _NS + s
        pltpu.sync_copy(z_hbm, acc.at[pl.ds(s * _RPT, _RPT)])
        pltpu.sync_copy(src_hbm.at[wid], src_v)
        pltpu.sync_copy(dst_hbm.at[wid], dst_v)
        plsc.subcore_barrier()

        def step(j, carry):
            pltpu.async_copy(x_hbm.at[src_v.at[j]], rows_v, sem).wait()
            pltpu.sync_copy(rows_v, acc.at[dst_v.at[j]], add=True)
            return carry

        lax.fori_loop(0, _C, step, 0)
        plsc.subcore_barrier()
        pltpu.sync_copy(acc.at[pl.ds(s * _RPT, _RPT)],
                        out_hbm.at[pl.ds(c * _NACC + s * _RPT, _RPT)])

    return k(x, srcp, dstp, zeros_tile)


def _mlp_body(scale_ref, x_ref, a_ref, b_ref, w1_ref, c1_ref, w2_ref,
              c2_ref, wl_ref, cl_ref, o_ref):
    h = x_ref[...] + a_ref[...] + b_ref[...]
    r = jnp.maximum(
        jnp.dot(h, w1_ref[...], preferred_element_type=jnp.float32)
        + c1_ref[...], 0.0)
    z = (jnp.dot(r, w2_ref[...], preferred_element_type=jnp.float32)
         + c2_ref[...])
    lin = (jnp.dot(x_ref[...], wl_ref[...],
                   preferred_element_type=jnp.float32) + cl_ref[...])
    o_ref[...] = z + scale_ref[0, 0] * lin


def _block_diag(w):
    k, a, b = w.shape
    out = jnp.zeros((k * a, k * b), w.dtype)
    for i in range(k):
        out = out.at[i * a:(i + 1) * a, i * b:(i + 1) * b].set(w[i])
    return out


def kernel(x, edge_index, edge_weights, edge_attr, with_linear, W1, b1, W2,
           b2, Wl, bl):
    src = edge_index[0]
    dst = edge_index[1]
    pad = _EPAD - _E
    per_tile = _E // _NW
    tpad = _C * _CH - per_tile
    psrc = jnp.zeros((_NW, tpad), jnp.int32)
    pdst = jnp.broadcast_to(
        _N + (jnp.arange(tpad, dtype=jnp.int32) % (_NACC - _N)),
        (_NW, tpad))
    srcp = jnp.concatenate([src.reshape(_NW, per_tile), psrc], axis=1)
    dstp = jnp.concatenate([dst.reshape(_NW, per_tile), pdst], axis=1)
    srcp = srcp.reshape(_NW, _C, _CH)
    dstp = dstp.reshape(_NW, _C, _CH)
    zeros_tile = jnp.zeros((_RPT, _D), jnp.float32)

    parts = _sc_scatter_add(x, srcp, dstp, zeros_tile)
    agg0 = parts[:_N]
    agg1 = parts[_NACC:_NACC + _N]

    bd1 = _block_diag(W1)
    bd2 = _block_diag(W2)
    bdl = _block_diag(Wl[:, :_DS, :_DS])
    c1 = b1.reshape(1, _D)
    c2 = b2.reshape(1, _D)
    cl = bl[:, :_DS].reshape(1, _D)
    scale = jnp.where(with_linear != 0, 1.0, 0.0).astype(
        jnp.float32).reshape(1, 1)

    blk = 1000
    grid = _N // blk
    full = pl.BlockSpec((_D, _D), lambda i: (0, 0))
    bias = pl.BlockSpec((1, _D), lambda i: (0, 0))
    rows = pl.BlockSpec((blk, _D), lambda i: (i, 0))
    out = pl.pallas_call(
        _mlp_body,
        grid=(grid,),
        in_specs=[pl.BlockSpec((1, 1), lambda i: (0, 0)),
                  rows, rows, rows, full, bias, full, bias, full, bias],
        out_specs=rows,
        out_shape=jax.ShapeDtypeStruct((_N, _D), jnp.float32),
    )(scale, x, agg0, agg1, bd1, c1, bd2, c2, bdl, cl)
    return out
